# baseline (speedup 1.0000x reference)
import dataclasses
import functools

import jax
import jax.numpy as jnp
from jax import lax
from jax.experimental import pallas as pl
from jax.experimental.pallas import tpu as pltpu
from jax.experimental.pallas import tpu_sc as plsc

N = 10000
E = 320000
D = 128
NEFF = 1000
B = 4096

NPAD = 10240
DROW = NPAD // 128
NW = 32
K = 80
NCH = 126
EPW = K * NCH
EPAD = NW * EPW
RPT = NPAD // 16
BPW = B // NW
BLK = 512
_PREC = lax.Precision.HIGHEST


_GDN = lax.GatherDimensionNumbers(offset_dims=(), collapsed_slice_dims=(0,),
                                  start_index_map=(0,))


def _permute(vec, idx16):
    return lax.gather(vec, idx16[:, None], _GDN, (1,),
                      mode=lax.GatherScatterMode.PROMISE_IN_BOUNDS)


def _sc_params():
    cp = pltpu.CompilerParams()
    if "needs_layout_passes" in pltpu.CompilerParams.__dataclass_fields__:
        cp = dataclasses.replace(cp, needs_layout_passes=False)
    return cp


def _tc_pre_body(x_ref, wn_ref, ws_ref, y_ref, s_ref):
    x = x_ref[...]
    y_ref[...] = lax.dot(x, wn_ref[...], precision=_PREC)
    s_ref[...] = lax.dot(x, ws_ref[...], precision=_PREC)


def _tc_pre(xp, wn, ws):
    return pl.pallas_call(
        _tc_pre_body,
        grid=(NPAD // BLK,),
        in_specs=[pl.BlockSpec((BLK, D), lambda i: (i, 0)),
                  pl.BlockSpec((D, D), lambda i: (0, 0)),
                  pl.BlockSpec((D, D), lambda i: (0, 0))],
        out_specs=[pl.BlockSpec((BLK, D), lambda i: (i, 0)),
                   pl.BlockSpec((BLK, D), lambda i: (i, 0))],
        out_shape=[jax.ShapeDtypeStruct((NPAD, D), jnp.float32),
                   jax.ShapeDtypeStruct((NPAD, D), jnp.float32)],
    )(xp, wn, ws)


def _tc_mid_body(acc_ref, dx_ref, s0_ref, wn_ref, ws_ref, y1_ref, s1_ref, inv_ref):
    den = dx_ref[0] + dx_ref[1]
    inv = 1.0 / jnp.maximum(den, 1e-12)
    agg = (acc_ref[0] + acc_ref[1]) * inv
    h1 = jnp.maximum(s0_ref[...] + agg, 0.0)
    y1_ref[...] = lax.dot(h1, wn_ref[...], precision=_PREC)
    s1_ref[...] = lax.dot(h1, ws_ref[...], precision=_PREC)
    inv_ref[...] = inv


def _tc_mid(acc0, denx, s0, wn1, ws1):
    return pl.pallas_call(
        _tc_mid_body,
        grid=(NPAD // BLK,),
        in_specs=[pl.BlockSpec((2, BLK, D), lambda i: (0, i, 0)),
                  pl.BlockSpec((2, BLK, D), lambda i: (0, i, 0)),
                  pl.BlockSpec((BLK, D), lambda i: (i, 0)),
                  pl.BlockSpec((D, D), lambda i: (0, 0)),
                  pl.BlockSpec((D, D), lambda i: (0, 0))],
        out_specs=[pl.BlockSpec((BLK, D), lambda i: (i, 0)),
                   pl.BlockSpec((BLK, D), lambda i: (i, 0)),
                   pl.BlockSpec((BLK, D), lambda i: (i, 0))],
        out_shape=[jax.ShapeDtypeStruct((NPAD, D), jnp.float32),
                   jax.ShapeDtypeStruct((NPAD, D), jnp.float32),
                   jax.ShapeDtypeStruct((NPAD, D), jnp.float32)],
    )(acc0, denx, s0, wn1, ws1)


def _tc_post_body(acc_ref, s1_ref, inv_ref, h2_ref):
    agg = (acc_ref[0] + acc_ref[1]) * inv_ref[...]
    h2_ref[...] = jnp.maximum(s1_ref[...] + agg, 0.0)


def _tc_post(acc1, s1, invb):
    return pl.pallas_call(
        _tc_post_body,
        grid=(NPAD // BLK,),
        in_specs=[pl.BlockSpec((2, BLK, D), lambda i: (0, i, 0)),
                  pl.BlockSpec((BLK, D), lambda i: (i, 0)),
                  pl.BlockSpec((BLK, D), lambda i: (i, 0))],
        out_specs=pl.BlockSpec((BLK, D), lambda i: (i, 0)),
        out_shape=jax.ShapeDtypeStruct((NPAD, D), jnp.float32),
    )(acc1, s1, invb)


def _make_sc_scatter(with_den, k, nch):
    mesh = plsc.VectorSubcoreMesh(core_axis_name="c", subcore_axis_name="s")
    epw = k * nch

    out_type = [jax.ShapeDtypeStruct((2, NPAD, D), jnp.float32)]
    scratch = [
        pltpu.VMEM((3, k), jnp.int32),
        pltpu.VMEM((3, k), jnp.int32),
        pltpu.VMEM((3, k), jnp.int32),
        pltpu.VMEM((k, D), jnp.float32),
        pltpu.VMEM((k, D), jnp.float32),
        pltpu.VMEM((k, D), jnp.float32),
        pltpu.VMEM_SHARED((NPAD, D), jnp.float32),
        pltpu.SemaphoreType.DMA((3,)),
        pltpu.SemaphoreType.DMA((3,)),
        pltpu.SemaphoreType.DMA((3,)),
    ]
    if with_den:
        out_type.append(jax.ShapeDtypeStruct((2, NPAD, D), jnp.float32))
        scratch += [
            pltpu.VMEM((DROW, 128), jnp.float32),
            pltpu.VMEM((DROW,), jnp.int32),
            pltpu.VMEM((DROW // 16, 128), jnp.float32),
            pltpu.VMEM_SHARED((DROW, 128), jnp.float32),
        ]

    @functools.partial(
        pl.kernel,
        out_type=out_type if with_den else out_type[0],
        mesh=mesh,
        compiler_params=_sc_params(),
        scratch_types=scratch,
    )
    def sc_scatter(y_hbm, ez_hbm, *refs):
        if with_den:
            (out_hbm, denx_hbm, z0, z1, z2,
             r0, r1, r2, acc_sh, zsem, gsem, ssem,
             den_v, ridx_v, dmy_v, den_sh) = refs
        else:
            (out_hbm, z0, z1, z2,
             r0, r1, r2, acc_sh, zsem, gsem, ssem) = refs
        z_b = [z0, z1, z2]
        rows_b = [r0, r1, r2]
        cid = lax.axis_index("c")
        sid = lax.axis_index("s")
        wid = cid * 16 + sid
        base = wid * epw
        zero16 = jnp.zeros((16,), jnp.float32)
        lane = lax.iota(jnp.int32, 16)
        mask0 = lane == 0

        @pl.loop(0, k)
        def _zero_rows(r):
            for c in range(D // 16):
                r0[r, pl.ds(c * 16, 16)] = zero16

        @pl.loop(0, RPT // 80)
        def _zero_acc(z):
            pltpu.sync_copy(r0.at[pl.ds(0, 80)],
                            acc_sh.at[pl.ds(sid * RPT + z * 80, 80)])

        if with_den:
            @pl.loop(0, DROW)
            def _zero_den(r):
                for c in range(128 // 16):
                    den_v[r, pl.ds(c * 16, 16)] = zero16
            for g in range(DROW // 16):
                ridx_v[pl.ds(g * 16, 16)] = lane + g * 16
            pltpu.sync_copy(r0.at[pl.ds(0, DROW // 16)],
                            den_sh.at[pl.ds(sid * (DROW // 16), DROW // 16)])

        plsc.subcore_barrier()

        def issue_idx(ci, p):
            pltpu.async_copy(ez_hbm.at[wid, ci], z_b[p], zsem.at[p])

        def wait_idx(p):
            pltpu.make_async_copy(ez_hbm.at[0, 0], z_b[p],
                                  zsem.at[p]).wait()

        def issue_gather(p):
            pltpu.async_copy(y_hbm.at[z_b[p].at[0]], rows_b[p], gsem.at[p])

        def wait_gather(p):
            pltpu.make_async_copy(y_hbm.at[pl.ds(0, k)], rows_b[p],
                                  gsem.at[p]).wait()

        def issue_scatter(p):
            pltpu.async_copy(rows_b[p], acc_sh.at[z_b[p].at[1]],
                             ssem.at[p], add=True)

        def wait_scatter(p):
            pltpu.make_async_copy(rows_b[p], acc_sh.at[z_b[p].at[1]],
                                  ssem.at[p]).wait()

        one16 = jnp.full((16,), 1, jnp.int32)
        two16 = jnp.full((16,), 2, jnp.int32)

        def scale(p):
            rv = rows_b[p]
            zv = z_b[p]

            @pl.loop(0, k, step=2)
            def _scale(j):
                for u in range(2):
                    jj = j + u
                    j16 = jnp.full((16,), jj, jnp.int32)
                    wspl = plsc.bitcast(
                        plsc.load_gather(zv, [two16, j16]), jnp.float32)
                    for c in range(D // 16):
                        sl = pl.ds(c * 16, 16)
                        rv[jj, sl] = rv[jj, sl] * wspl
                    if with_den:
                        dspl = plsc.load_gather(zv, [one16, j16])
                        plsc.addupdate_scatter(
                            den_v,
                            [lax.shift_right_logical(dspl, 7),
                             jnp.bitwise_and(dspl, 127)],
                            wspl, mask=mask0)

        def phase(ci, p, g1, wprev, i2):
            if g1:
                pn = (p + 1) % 3
                wait_idx(pn)
                issue_gather(pn)
            wait_gather(p)
            scale(p)
            issue_scatter(p)
            if wprev:
                wait_scatter((p + 2) % 3)
            if i2 is not None:
                issue_idx(i2, (p + 2) % 3)

        issue_idx(0, 0)
        issue_idx(1, 1)
        wait_idx(0)
        issue_gather(0)

        phase(0, 0, True, False, 2)
        phase(1, 1, True, True, 3)
        phase(2, 2, True, True, 4)

        @pl.loop(1, (nch - 6) // 3 + 1)
        def _steady(g):
            ci = g * 3
            phase(ci, 0, True, True, ci + 2)
            phase(ci + 1, 1, True, True, ci + 3)
            phase(ci + 2, 2, True, True, ci + 4)

        phase(nch - 3, 0, True, True, nch - 1)
        phase(nch - 2, 1, True, True, None)
        phase(nch - 1, 2, False, True, None)
        wait_scatter((nch - 1) % 3)

        if with_den:
            pltpu.sync_copy(den_v, den_sh.at[ridx_v], add=True)

        plsc.subcore_barrier()
        pltpu.sync_copy(acc_sh.at[pl.ds(sid * RPT, RPT)],
                        out_hbm.at[cid, pl.ds(sid * RPT, RPT)])

        if with_den:
            nd = DROW // 16
            pltpu.sync_copy(den_sh.at[pl.ds(sid * nd, nd)], dmy_v)

            @pl.loop(0, RPT // 80)
            def _expand(z):
                @pl.loop(0, 80)
                def _row(j):
                    nl = z * 80 + j
                    hi = jnp.full((16,), lax.shift_right_logical(nl, 7),
                                  jnp.int32)
                    lo = jnp.full((16,), jnp.bitwise_and(nl, 127), jnp.int32)
                    dspl = plsc.load_gather(dmy_v, [hi, lo])
                    for c in range(D // 16):
                        r0[j, pl.ds(c * 16, 16)] = dspl

                pltpu.sync_copy(
                    r0.at[pl.ds(0, 80)],
                    denx_hbm.at[cid, pl.ds(sid * RPT + z * 80, 80)])

    return sc_scatter


_sc_scatter0 = _make_sc_scatter(True, 88, 114)
_sc_scatter1 = _make_sc_scatter(False, 120, 84)


def _make_sc_lookup():
    mesh = plsc.VectorSubcoreMesh(core_axis_name="c", subcore_axis_name="s")

    @functools.partial(
        pl.kernel,
        out_type=jax.ShapeDtypeStruct((B, D), jnp.float32),
        mesh=mesh,
        scratch_types=[
            pltpu.VMEM((BPW,), jnp.int32),
            pltpu.VMEM((BPW,), jnp.int32),
            pltpu.VMEM((BPW, D), jnp.float32),
            pltpu.VMEM((BPW, D), jnp.float32),
            pltpu.SemaphoreType.DMA,
        ],
    )
    def sc_lookup(h_hbm, emb_hbm, xn_hbm, eff_hbm, out_hbm,
                  i1_v, i2_v, a_v, b_v, sem):
        wid = lax.axis_index("c") * 16 + lax.axis_index("s")
        bb = wid * BPW
        pltpu.sync_copy(xn_hbm.at[pl.ds(bb, BPW)], i1_v)
        pltpu.sync_copy(eff_hbm.at[pl.ds(bb, BPW)], i2_v)
        cp1 = pltpu.async_copy(h_hbm.at[i1_v], a_v, sem)
        cp2 = pltpu.async_copy(emb_hbm.at[i2_v], b_v, sem)
        cp1.wait()
        cp2.wait()

        @pl.loop(0, BPW)
        def _add(r):
            for c in range(D // 16):
                sl = pl.ds(c * 16, 16)
                a_v[r, sl] = a_v[r, sl] + b_v[r, sl]

        pltpu.sync_copy(a_v, out_hbm.at[pl.ds(bb, BPW)])

    return sc_lookup


_sc_lookup = _make_sc_lookup()


def kernel(graph_x, edge_index, x_nodes, effect_ids, chemical_similarity,
           effect_emb, W_self0, W_neigh0, W_self1, W_neigh1):
    src = jnp.pad(edge_index[0], (0, EPAD - E))
    dst = jnp.pad(edge_index[1], (0, EPAD - E))
    wi = jnp.pad(lax.bitcast_convert_type(chemical_similarity, jnp.int32),
                 (0, EPAD - E))
    xp = jnp.pad(graph_x, ((0, NPAD - N), (0, 0)))

    def zip_edges(k, nch):
        tot = NW * k * nch
        z = jnp.stack([src[:tot], dst[:tot], wi[:tot]])
        return z.reshape(3, NW, nch, k).transpose(1, 2, 0, 3)

    y0, s0 = _tc_pre(xp, W_neigh0, W_self0)
    acc0, denx = _sc_scatter0(y0, zip_edges(88, 114))
    y1, s1, invb = _tc_mid(acc0, denx, s0, W_neigh1, W_self1)
    acc1 = _sc_scatter1(y1, zip_edges(120, 84))
    h2 = _tc_post(acc1, s1, invb)
    return _sc_lookup(h2, effect_emb, x_nodes, effect_ids)


# speedup vs baseline: 1.0823x; 1.0823x over previous
import dataclasses
import functools

import jax
import jax.numpy as jnp
from jax import lax
from jax.experimental import pallas as pl
from jax.experimental.pallas import tpu as pltpu
from jax.experimental.pallas import tpu_sc as plsc

N = 10000
E = 320000
D = 128
NEFF = 1000
B = 4096

NPAD = 10240
DROW = NPAD // 128
NW = 32
K = 80
NCH = 126
EPW = K * NCH
EPAD = NW * EPW
RPT = NPAD // 16
BPW = B // NW
BLK = 512
_PREC = lax.Precision.HIGHEST


_GDN = lax.GatherDimensionNumbers(offset_dims=(), collapsed_slice_dims=(0,),
                                  start_index_map=(0,))


def _permute(vec, idx16):
    return lax.gather(vec, idx16[:, None], _GDN, (1,),
                      mode=lax.GatherScatterMode.PROMISE_IN_BOUNDS)


def _sc_params():
    cp = pltpu.CompilerParams()
    if "needs_layout_passes" in pltpu.CompilerParams.__dataclass_fields__:
        cp = dataclasses.replace(cp, needs_layout_passes=False)
    return cp


def _tc_pre_body(x_ref, wn_ref, ws_ref, y_ref, s_ref):
    x = x_ref[...]
    y_ref[...] = lax.dot(x, wn_ref[...], precision=_PREC)
    s_ref[...] = lax.dot(x, ws_ref[...], precision=_PREC)


def _tc_pre(xp, wn, ws):
    return pl.pallas_call(
        _tc_pre_body,
        grid=(NPAD // BLK,),
        in_specs=[pl.BlockSpec((BLK, D), lambda i: (i, 0)),
                  pl.BlockSpec((D, D), lambda i: (0, 0)),
                  pl.BlockSpec((D, D), lambda i: (0, 0))],
        out_specs=[pl.BlockSpec((BLK, D), lambda i: (i, 0)),
                   pl.BlockSpec((BLK, D), lambda i: (i, 0))],
        out_shape=[jax.ShapeDtypeStruct((NPAD, D), jnp.float32),
                   jax.ShapeDtypeStruct((NPAD, D), jnp.float32)],
    )(xp, wn, ws)


def _tc_mid_body(acc_ref, dx_ref, s0_ref, wn_ref, ws_ref, y1_ref, s1_ref, inv_ref):
    den = dx_ref[0] + dx_ref[1]
    inv = 1.0 / jnp.maximum(den, 1e-12)
    agg = (acc_ref[0] + acc_ref[1]) * inv
    h1 = jnp.maximum(s0_ref[...] + agg, 0.0)
    y1_ref[...] = lax.dot(h1, wn_ref[...], precision=_PREC)
    s1_ref[...] = lax.dot(h1, ws_ref[...], precision=_PREC)
    inv_ref[...] = inv


def _tc_mid(acc0, denx, s0, wn1, ws1):
    return pl.pallas_call(
        _tc_mid_body,
        grid=(NPAD // BLK,),
        in_specs=[pl.BlockSpec((2, BLK, D), lambda i: (0, i, 0)),
                  pl.BlockSpec((2, BLK, D), lambda i: (0, i, 0)),
                  pl.BlockSpec((BLK, D), lambda i: (i, 0)),
                  pl.BlockSpec((D, D), lambda i: (0, 0)),
                  pl.BlockSpec((D, D), lambda i: (0, 0))],
        out_specs=[pl.BlockSpec((BLK, D), lambda i: (i, 0)),
                   pl.BlockSpec((BLK, D), lambda i: (i, 0)),
                   pl.BlockSpec((BLK, D), lambda i: (i, 0))],
        out_shape=[jax.ShapeDtypeStruct((NPAD, D), jnp.float32),
                   jax.ShapeDtypeStruct((NPAD, D), jnp.float32),
                   jax.ShapeDtypeStruct((NPAD, D), jnp.float32)],
    )(acc0, denx, s0, wn1, ws1)


def _tc_post_body(acc_ref, s1_ref, inv_ref, h2_ref):
    agg = (acc_ref[0] + acc_ref[1]) * inv_ref[...]
    h2_ref[...] = jnp.maximum(s1_ref[...] + agg, 0.0)


def _tc_post(acc1, s1, invb):
    return pl.pallas_call(
        _tc_post_body,
        grid=(NPAD // BLK,),
        in_specs=[pl.BlockSpec((2, BLK, D), lambda i: (0, i, 0)),
                  pl.BlockSpec((BLK, D), lambda i: (i, 0)),
                  pl.BlockSpec((BLK, D), lambda i: (i, 0))],
        out_specs=pl.BlockSpec((BLK, D), lambda i: (i, 0)),
        out_shape=jax.ShapeDtypeStruct((NPAD, D), jnp.float32),
    )(acc1, s1, invb)


def _make_sc_scatter(with_den, k, nch):
    mesh = plsc.VectorSubcoreMesh(core_axis_name="c", subcore_axis_name="s")
    epw = k * nch

    out_type = [jax.ShapeDtypeStruct((2, NPAD, D), jnp.float32)]
    scratch = [
        pltpu.VMEM((k,), jnp.int32),
        pltpu.VMEM((k,), jnp.int32),
        pltpu.VMEM((k,), jnp.int32),
        pltpu.VMEM((k,), jnp.int32),
        pltpu.VMEM((k,), jnp.int32),
        pltpu.VMEM((k,), jnp.int32),
        pltpu.VMEM((k,), jnp.float32),
        pltpu.VMEM((k,), jnp.float32),
        pltpu.VMEM((k,), jnp.float32),
        pltpu.VMEM((k, D), jnp.float32),
        pltpu.VMEM((k, D), jnp.float32),
        pltpu.VMEM((k, D), jnp.float32),
        pltpu.VMEM_SHARED((NPAD, D), jnp.float32),
        pltpu.SemaphoreType.DMA((3,)),
        pltpu.SemaphoreType.DMA((3,)),
        pltpu.SemaphoreType.DMA((3,)),
        pltpu.SemaphoreType.DMA((3,)),
        pltpu.SemaphoreType.DMA((3,)),
    ]
    if with_den:
        out_type.append(jax.ShapeDtypeStruct((2, NPAD, D), jnp.float32))
        scratch += [
            pltpu.VMEM((DROW, 128), jnp.float32),
            pltpu.VMEM((DROW,), jnp.int32),
            pltpu.VMEM((DROW // 16, 128), jnp.float32),
            pltpu.VMEM_SHARED((DROW, 128), jnp.float32),
        ]

    @functools.partial(
        pl.kernel,
        out_type=out_type if with_den else out_type[0],
        mesh=mesh,
        compiler_params=_sc_params(),
        scratch_types=scratch,
    )
    def sc_scatter(y_hbm, src_hbm, dst_hbm, w_hbm, *refs):
        if with_den:
            (out_hbm, denx_hbm, s0, s1, s2, d0, d1, d2, w0, w1, w2,
             r0, r1, r2, acc_sh, srcsem, dsem, wsem, gsem, ssem,
             den_v, ridx_v, dmy_v, den_sh) = refs
        else:
            (out_hbm, s0, s1, s2, d0, d1, d2, w0, w1, w2,
             r0, r1, r2, acc_sh, srcsem, dsem, wsem, gsem, ssem) = refs
        src_b = [s0, s1, s2]
        dst_b = [d0, d1, d2]
        w_b = [w0, w1, w2]
        rows_b = [r0, r1, r2]
        cid = lax.axis_index("c")
        sid = lax.axis_index("s")
        wid = cid * 16 + sid
        base = wid * epw
        zero16 = jnp.zeros((16,), jnp.float32)
        lane = lax.iota(jnp.int32, 16)
        mask0 = lane == 0

        @pl.loop(0, k)
        def _zero_rows(r):
            for c in range(D // 16):
                r0[r, pl.ds(c * 16, 16)] = zero16

        @pl.loop(0, RPT // 80)
        def _zero_acc(z):
            pltpu.sync_copy(r0.at[pl.ds(0, 80)],
                            acc_sh.at[pl.ds(sid * RPT + z * 80, 80)])

        if with_den:
            @pl.loop(0, DROW)
            def _zero_den(r):
                for c in range(128 // 16):
                    den_v[r, pl.ds(c * 16, 16)] = zero16
            for g in range(DROW // 16):
                ridx_v[pl.ds(g * 16, 16)] = lane + g * 16
            pltpu.sync_copy(r0.at[pl.ds(0, DROW // 16)],
                            den_sh.at[pl.ds(sid * (DROW // 16), DROW // 16)])

        plsc.subcore_barrier()

        def issue_idx(ci, p):
            off = base + ci * k
            pltpu.async_copy(src_hbm.at[pl.ds(off, k)], src_b[p], srcsem.at[p])
            pltpu.async_copy(dst_hbm.at[pl.ds(off, k)], dst_b[p], dsem.at[p])
            pltpu.async_copy(w_hbm.at[pl.ds(off, k)], w_b[p], wsem.at[p])

        def wait_idx(p):
            pltpu.make_async_copy(src_hbm.at[pl.ds(0, k)], src_b[p],
                                  srcsem.at[p]).wait()
            pltpu.make_async_copy(dst_hbm.at[pl.ds(0, k)], dst_b[p],
                                  dsem.at[p]).wait()
            pltpu.make_async_copy(w_hbm.at[pl.ds(0, k)], w_b[p],
                                  wsem.at[p]).wait()

        def issue_gather(p):
            pltpu.async_copy(y_hbm.at[src_b[p]], rows_b[p], gsem.at[p])

        def wait_gather(p):
            pltpu.make_async_copy(y_hbm.at[pl.ds(0, k)], rows_b[p],
                                  gsem.at[p]).wait()

        def issue_scatter(p):
            pltpu.async_copy(rows_b[p], acc_sh.at[dst_b[p]],
                             ssem.at[p], add=True)

        def wait_scatter(p):
            pltpu.make_async_copy(rows_b[p], acc_sh.at[dst_b[p]],
                                  ssem.at[p]).wait()

        def scale(p):
            rv = rows_b[p]
            wv = w_b[p]
            dv = dst_b[p]

            @pl.loop(0, k, step=2)
            def _scale(j):
                for u in range(2):
                    jj = j + u
                    j16 = jnp.full((16,), jj, jnp.int32)
                    wspl = plsc.load_gather(wv, [j16])
                    for c in range(D // 16):
                        sl = pl.ds(c * 16, 16)
                        rv[jj, sl] = rv[jj, sl] * wspl
                    if with_den:
                        dspl = plsc.load_gather(dv, [j16])
                        plsc.addupdate_scatter(
                            den_v,
                            [lax.shift_right_logical(dspl, 7),
                             jnp.bitwise_and(dspl, 127)],
                            wspl, mask=mask0)

        def phase(ci, p, g1, wprev, i2):
            if g1:
                pn = (p + 1) % 3
                wait_idx(pn)
                issue_gather(pn)
            wait_gather(p)
            scale(p)
            issue_scatter(p)
            if wprev:
                wait_scatter((p + 2) % 3)
            if i2 is not None:
                issue_idx(i2, (p + 2) % 3)

        issue_idx(0, 0)
        issue_idx(1, 1)
        wait_idx(0)
        issue_gather(0)

        phase(0, 0, True, False, 2)
        phase(1, 1, True, True, 3)
        phase(2, 2, True, True, 4)

        @pl.loop(1, (nch - 6) // 3 + 1)
        def _steady(g):
            ci = g * 3
            phase(ci, 0, True, True, ci + 2)
            phase(ci + 1, 1, True, True, ci + 3)
            phase(ci + 2, 2, True, True, ci + 4)

        phase(nch - 3, 0, True, True, nch - 1)
        phase(nch - 2, 1, True, True, None)
        phase(nch - 1, 2, False, True, None)
        wait_scatter((nch - 1) % 3)

        if with_den:
            pltpu.sync_copy(den_v, den_sh.at[ridx_v], add=True)

        plsc.subcore_barrier()
        pltpu.sync_copy(acc_sh.at[pl.ds(sid * RPT, RPT)],
                        out_hbm.at[cid, pl.ds(sid * RPT, RPT)])

        if with_den:
            nd = DROW // 16
            pltpu.sync_copy(den_sh.at[pl.ds(sid * nd, nd)], dmy_v)

            @pl.loop(0, RPT // 80)
            def _expand(z):
                @pl.loop(0, 80)
                def _row(j):
                    nl = z * 80 + j
                    hi = jnp.full((16,), lax.shift_right_logical(nl, 7),
                                  jnp.int32)
                    lo = jnp.full((16,), jnp.bitwise_and(nl, 127), jnp.int32)
                    dspl = plsc.load_gather(dmy_v, [hi, lo])
                    for c in range(D // 16):
                        r0[j, pl.ds(c * 16, 16)] = dspl

                pltpu.sync_copy(
                    r0.at[pl.ds(0, 80)],
                    denx_hbm.at[cid, pl.ds(sid * RPT + z * 80, 80)])

    return sc_scatter


_sc_scatter0 = _make_sc_scatter(True, 88, 114)
_sc_scatter1 = _make_sc_scatter(False, 120, 84)


def _make_sc_lookup():
    mesh = plsc.VectorSubcoreMesh(core_axis_name="c", subcore_axis_name="s")

    @functools.partial(
        pl.kernel,
        out_type=jax.ShapeDtypeStruct((B, D), jnp.float32),
        mesh=mesh,
        scratch_types=[
            pltpu.VMEM((BPW,), jnp.int32),
            pltpu.VMEM((BPW,), jnp.int32),
            pltpu.VMEM((BPW, D), jnp.float32),
            pltpu.VMEM((BPW, D), jnp.float32),
            pltpu.SemaphoreType.DMA,
        ],
    )
    def sc_lookup(h_hbm, emb_hbm, xn_hbm, eff_hbm, out_hbm,
                  i1_v, i2_v, a_v, b_v, sem):
        wid = lax.axis_index("c") * 16 + lax.axis_index("s")
        bb = wid * BPW
        pltpu.sync_copy(xn_hbm.at[pl.ds(bb, BPW)], i1_v)
        pltpu.sync_copy(eff_hbm.at[pl.ds(bb, BPW)], i2_v)
        cp1 = pltpu.async_copy(h_hbm.at[i1_v], a_v, sem)
        cp2 = pltpu.async_copy(emb_hbm.at[i2_v], b_v, sem)
        cp1.wait()
        cp2.wait()

        @pl.loop(0, BPW)
        def _add(r):
            for c in range(D // 16):
                sl = pl.ds(c * 16, 16)
                a_v[r, sl] = a_v[r, sl] + b_v[r, sl]

        pltpu.sync_copy(a_v, out_hbm.at[pl.ds(bb, BPW)])

    return sc_lookup


_sc_lookup = _make_sc_lookup()


def kernel(graph_x, edge_index, x_nodes, effect_ids, chemical_similarity,
           effect_emb, W_self0, W_neigh0, W_self1, W_neigh1):
    src = jnp.pad(edge_index[0], (0, EPAD - E))
    dst = jnp.pad(edge_index[1], (0, EPAD - E))
    w = jnp.pad(chemical_similarity, (0, EPAD - E))
    xp = jnp.pad(graph_x, ((0, NPAD - N), (0, 0)))

    y0, s0 = _tc_pre(xp, W_neigh0, W_self0)
    acc0, denx = _sc_scatter0(y0, src, dst, w)
    y1, s1, invb = _tc_mid(acc0, denx, s0, W_neigh1, W_self1)
    acc1 = _sc_scatter1(y1, src, dst, w)
    h2 = _tc_post(acc1, s1, invb)
    return _sc_lookup(h2, effect_emb, x_nodes, effect_ids)
